# baseline (speedup 1.0000x reference)
import functools

import jax
import jax.numpy as jnp
from jax import lax
from jax.experimental import pallas as pl
from jax.experimental.pallas import tpu as pltpu
from jax.experimental.pallas import tpu_sc as plsc

_BATCH = 16384
_OUT_DIM = 32
_NC = 2
_NS = 16
_NW = _NC * _NS
_B_PER_W = _BATCH // _NW
_G = 16
_N_GROUPS = _B_PER_W // _G
_L = 16


def _embed_body(cls_hbm, tabT_hbm, outT_hbm, idx_v, ring_v, stage_v, sem):
    wid = lax.axis_index("s") * _NC + lax.axis_index("c")
    base = wid * _B_PER_W
    pltpu.sync_copy(cls_hbm.at[pl.ds(base, _B_PER_W)], idx_v)

    lanes = lax.iota(jnp.int32, _L)
    rows_lo = lanes
    rows_hi = _L + lanes

    def fire(g, u):
        vec = idx_v[pl.ds(g * _G, _G)]
        vv = pl.multiple_of((vec[u] >> 7) * 128, 128)
        pltpu.async_copy(
            tabT_hbm.at[:, pl.ds(vv, 128)],
            ring_v.at[u],
            sem.at[u],
        )

    def extract(g, u, c_vec):
        c = jnp.full((_L,), c_vec[u], jnp.int32)
        pos = jnp.full((_L,), g * _G + u, jnp.int32)
        lo = plsc.load_gather(ring_v.at[u], [rows_lo, c])
        hi = plsc.load_gather(ring_v.at[u], [rows_hi, c])
        plsc.store_scatter(stage_v, [rows_lo, pos], lo)
        plsc.store_scatter(stage_v, [rows_hi, pos], hi)

    def wait(u):
        pltpu.make_async_copy(
            tabT_hbm.at[:, pl.ds(0, 128)], ring_v.at[u], sem.at[u]
        ).wait()

    for u in range(_G):
        fire(0, u)

    def group(g, c_prev):
        vec = idx_v[pl.ds(g * _G, _G)]
        for u in range(_G):
            wait(u)
            extract(g - 1, u, c_prev)
            fire(g, u)
        return vec & 127

    c0 = idx_v[pl.ds(0, _G)] & 127
    c_last = lax.fori_loop(1, _N_GROUPS, group, c0)

    for u in range(_G):
        wait(u)
        extract(_N_GROUPS - 1, u, c_last)

    pltpu.sync_copy(stage_v, outT_hbm.at[:, pl.ds(base, _B_PER_W)])


@jax.jit
def kernel(cls, table):
    mesh = plsc.VectorSubcoreMesh(core_axis_name="c", subcore_axis_name="s")
    run = functools.partial(
        pl.kernel,
        mesh=mesh,
        out_type=jax.ShapeDtypeStruct((_OUT_DIM, _BATCH), jnp.float32),
        scratch_types=[
            pltpu.VMEM((_B_PER_W,), jnp.int32),
            pltpu.VMEM((_G, _OUT_DIM, 128), jnp.float32),
            pltpu.VMEM((_OUT_DIM, _B_PER_W), jnp.float32),
            pltpu.SemaphoreType.DMA((_G,)),
        ],
        compiler_params=pltpu.CompilerParams(needs_layout_passes=False),
    )(_embed_body)
    outT = run(cls.astype(jnp.int32), table.T)
    return outT.T


# speedup vs baseline: 1.0125x; 1.0125x over previous
import functools

import jax
import jax.numpy as jnp
from jax import lax
from jax.experimental import pallas as pl
from jax.experimental.pallas import tpu as pltpu
from jax.experimental.pallas import tpu_sc as plsc

_BATCH = 16384
_OUT_DIM = 32
_NC = 2
_NS = 16
_NW = _NC * _NS
_B_PER_W = _BATCH // _NW
_G = 16
_N_GROUPS = _B_PER_W // _G
_L = 16


def _embed_body(cls_hbm, tabT_hbm, outT_hbm, idx_v, ring_v, stage_v, sem):
    wid = lax.axis_index("s") * _NC + lax.axis_index("c")
    base = wid * _B_PER_W
    pltpu.sync_copy(cls_hbm.at[pl.ds(base, _B_PER_W)], idx_v)

    lanes = lax.iota(jnp.int32, _L)
    rows_lo = lanes
    rows_hi = _L + lanes

    def fire(g, u):
        vec = idx_v[pl.ds(g * _G, _G)]
        vv = pl.multiple_of((vec[u] >> 7) * 128, 128)
        pltpu.async_copy(
            tabT_hbm.at[pl.ds(0, _L), pl.ds(vv, 128)],
            ring_v.at[u, pl.ds(0, _L)],
            sem.at[u],
        )
        pltpu.async_copy(
            tabT_hbm.at[pl.ds(_L, _L), pl.ds(vv, 128)],
            ring_v.at[u, pl.ds(_L, _L)],
            sem.at[u],
        )

    def extract(g, u, c_vec):
        c = jnp.full((_L,), c_vec[u], jnp.int32)
        pos = jnp.full((_L,), g * _G + u, jnp.int32)
        lo = plsc.load_gather(ring_v.at[u], [rows_lo, c])
        hi = plsc.load_gather(ring_v.at[u], [rows_hi, c])
        plsc.store_scatter(stage_v, [rows_lo, pos], lo)
        plsc.store_scatter(stage_v, [rows_hi, pos], hi)

    def wait(u):
        pltpu.make_async_copy(
            tabT_hbm.at[:, pl.ds(0, 128)], ring_v.at[u], sem.at[u]
        ).wait()

    for u in range(_G):
        fire(0, u)

    def group(g, c_prev):
        vec = idx_v[pl.ds(g * _G, _G)]
        for u in range(_G):
            wait(u)
            extract(g - 1, u, c_prev)
            fire(g, u)
        return vec & 127

    c0 = idx_v[pl.ds(0, _G)] & 127
    c_last = lax.fori_loop(1, _N_GROUPS, group, c0)

    for u in range(_G):
        wait(u)
        extract(_N_GROUPS - 1, u, c_last)

    pltpu.sync_copy(stage_v, outT_hbm.at[:, pl.ds(base, _B_PER_W)])


@jax.jit
def kernel(cls, table):
    mesh = plsc.VectorSubcoreMesh(core_axis_name="c", subcore_axis_name="s")
    run = functools.partial(
        pl.kernel,
        mesh=mesh,
        out_type=jax.ShapeDtypeStruct((_OUT_DIM, _BATCH), jnp.float32),
        scratch_types=[
            pltpu.VMEM((_B_PER_W,), jnp.int32),
            pltpu.VMEM((_G, _OUT_DIM, 128), jnp.float32),
            pltpu.VMEM((_OUT_DIM, _B_PER_W), jnp.float32),
            pltpu.SemaphoreType.DMA((_G,)),
        ],
        compiler_params=pltpu.CompilerParams(needs_layout_passes=False),
    )(_embed_body)
    outT = run(cls.astype(jnp.int32), table.T)
    return outT.T
